# baseline (speedup 1.0000x reference)
import functools

import jax
import jax.numpy as jnp
from jax import lax
from jax.experimental import pallas as pl
from jax.experimental.pallas import tpu as pltpu
from jax.experimental.pallas import tpu_sc as plsc

_DEPTH = 1000
_BATCH = 16384

_NC = 2
_NS = 16
_NW = _NC * _NS
_LANES = 16

_ROWS_PER_W = _BATCH // _NW
_BLK = 32
_CHUNKS = _BLK // _LANES
_NBLK = _ROWS_PER_W // _BLK
_BLK_WORDS = _BLK * _DEPTH


_NBUF = 3


def _onehot_body(idx_hbm, out_hbm, idx_v, b0, b1, b2, s0, s1, s2):
    bufs = [b0, b1, b2]
    sems = [s0, s1, s2]
    wid = lax.axis_index("s") * _NC + lax.axis_index("c")

    zeros16 = jnp.zeros((_LANES,), jnp.float32)

    for buf in bufs:
        def _zero(i, _, buf=buf):
            buf[pl.ds(i * _LANES, _LANES)] = zeros16
            return _

        lax.fori_loop(0, _BLK_WORDS // _LANES, _zero, None)

    pltpu.sync_copy(idx_hbm.at[pl.ds(wid * _ROWS_PER_W, _ROWS_PER_W)], idx_v)

    ones16 = jnp.ones((_LANES,), jnp.float32)
    lane = lax.iota(jnp.int32, _LANES)
    out_base = wid * _ROWS_PER_W * _DEPTH

    handles = [None] * _NBUF
    pending = [None] * _NBUF
    for blk in range(_NBLK):
        b = blk % _NBUF
        if handles[b] is not None:
            handles[b].wait()
            for pos in pending[b]:
                plsc.store_scatter(bufs[b], [pos], zeros16)
        positions = []
        for c in range(_CHUNKS):
            ids = idx_v[pl.ds(blk * _BLK + c * _LANES, _LANES)]
            pos = (c * _LANES + lane) * _DEPTH + ids
            plsc.store_scatter(bufs[b], [pos], ones16)
            positions.append(pos)
        pending[b] = positions
        handles[b] = pltpu.async_copy(
            bufs[b],
            out_hbm.at[pl.ds(out_base + blk * _BLK_WORDS, _BLK_WORDS)],
            sems[b],
        )
    for h in handles:
        if h is not None:
            h.wait()


@jax.jit
def _onehot_sc(X_in):
    mesh = plsc.VectorSubcoreMesh(core_axis_name="c", subcore_axis_name="s")
    fn = functools.partial(
        pl.kernel,
        mesh=mesh,
        out_type=jax.ShapeDtypeStruct((_BATCH * _DEPTH,), jnp.float32),
        scratch_types=[
            pltpu.VMEM((_ROWS_PER_W,), jnp.int32),
        ]
        + [pltpu.VMEM((_BLK_WORDS,), jnp.float32) for _ in range(_NBUF)]
        + [pltpu.SemaphoreType.DMA for _ in range(_NBUF)],
        compiler_params=pltpu.CompilerParams(needs_layout_passes=False),
    )(_onehot_body)
    return fn(X_in)


def kernel(X_in, ones):
    del ones
    return _onehot_sc(X_in).reshape(_BATCH, _DEPTH)


# speedup vs baseline: 1.0576x; 1.0576x over previous
import functools

import jax
import jax.numpy as jnp
from jax import lax
from jax.experimental import pallas as pl
from jax.experimental.pallas import tpu as pltpu
from jax.experimental.pallas import tpu_sc as plsc

_DEPTH = 1000
_BATCH = 16384

_NC = 2
_NS = 16
_NW = _NC * _NS
_LANES = 16

_ROWS_PER_W = _BATCH // _NW
_BLK = 32
_CHUNKS = _BLK // _LANES
_NBLK = _ROWS_PER_W // _BLK
_BLK_WORDS = _BLK * _DEPTH


_IDX_ROWS = _ROWS_PER_W // 128


def _onehot_body(idx_hbm, out_hbm, idx_v, zbuf, ones_v, pos_v, zsem, ssem):
    wid = lax.axis_index("s") * _NC + lax.axis_index("c")
    out_base = wid * _ROWS_PER_W * _DEPTH

    zeros16 = jnp.zeros((_LANES,), jnp.float32)
    ones16 = jnp.ones((_LANES,), jnp.float32)

    def _zfill(i, _):
        for u in range(8):
            zbuf[pl.ds((i * 8 + u) * _LANES, _LANES)] = zeros16
        return _

    lax.fori_loop(0, _BLK_WORDS // (_LANES * 8), _zfill, None)
    for u in range(128 // _LANES):
        ones_v[pl.ds(u * _LANES, _LANES)] = ones16

    pltpu.sync_copy(idx_hbm.at[pl.ds(wid * _ROWS_PER_W, _ROWS_PER_W)], idx_v)
    lane = lax.iota(jnp.int32, _LANES)
    for c in range(_ROWS_PER_W // _LANES):
        ids = idx_v[pl.ds(c * _LANES, _LANES)]
        pos = out_base + (c * _LANES + lane) * _DEPTH + ids
        pos_v[c // 8, pl.ds((c % 8) * _LANES, _LANES)] = pos

    zh = [
        pltpu.async_copy(
            zbuf,
            out_hbm.at[pl.ds(out_base + blk * _BLK_WORDS, _BLK_WORDS)],
            zsem,
        )
        for blk in range(_NBLK)
    ]
    for h in zh:
        h.wait()

    sh = [
        pltpu.async_copy(ones_v, out_hbm.at[pos_v.at[j]], ssem)
        for j in range(_IDX_ROWS)
    ]
    for h in sh:
        h.wait()


@jax.jit
def _onehot_sc(X_in):
    mesh = plsc.VectorSubcoreMesh(core_axis_name="c", subcore_axis_name="s")
    fn = functools.partial(
        pl.kernel,
        mesh=mesh,
        out_type=jax.ShapeDtypeStruct((_BATCH * _DEPTH,), jnp.float32),
        scratch_types=[
            pltpu.VMEM((_ROWS_PER_W,), jnp.int32),
            pltpu.VMEM((_BLK_WORDS,), jnp.float32),
            pltpu.VMEM((128,), jnp.float32),
            pltpu.VMEM((_IDX_ROWS, 128), jnp.int32),
            pltpu.SemaphoreType.DMA,
            pltpu.SemaphoreType.DMA,
        ],
        compiler_params=pltpu.CompilerParams(needs_layout_passes=False),
    )(_onehot_body)
    return fn(X_in)


def kernel(X_in, ones):
    del ones
    return _onehot_sc(X_in).reshape(_BATCH, _DEPTH)


# speedup vs baseline: 1.8640x; 1.7625x over previous
import functools

import jax
import jax.numpy as jnp
from jax import lax
from jax.experimental import pallas as pl
from jax.experimental.pallas import tpu as pltpu
from jax.experimental.pallas import tpu_sc as plsc

_DEPTH = 1000
_BATCH = 16384

_NC = 2
_NS = 16
_NW = _NC * _NS
_LANES = 16

_ROWS_PER_W = _BATCH // _NW
_BLK = 16
_NBLK = _ROWS_PER_W // _BLK
_NBUF = 3
_FULL = _DEPTH // _LANES
_TAIL = _DEPTH - _LANES


def _onehot_body(idx_hbm, out_hbm, idx_v, b0, b1, b2, s0, s1, s2):
    bufs = [b0, b1, b2]
    sems = [s0, s1, s2]
    wid = lax.axis_index("s") * _NC + lax.axis_index("c")
    row_base = wid * _ROWS_PER_W

    zeros16 = jnp.zeros((_LANES,), jnp.float32)
    for buf in bufs:
        def _zrow(r, _, buf=buf):
            for u in range(_FULL):
                buf[r, pl.ds(u * _LANES, _LANES)] = zeros16
            buf[r, pl.ds(_TAIL, _LANES)] = zeros16
            return _

        lax.fori_loop(0, _BLK, _zrow, None)

    pltpu.sync_copy(idx_hbm.at[pl.ds(row_base, _ROWS_PER_W)], idx_v)

    ones16 = jnp.ones((_LANES,), jnp.float32)
    lane = lax.iota(jnp.int32, _LANES)

    handles = [None] * _NBUF
    pending = [None] * _NBUF
    for blk in range(_NBLK):
        b = blk % _NBUF
        if handles[b] is not None:
            handles[b].wait()
            plsc.store_scatter(bufs[b], [lane, pending[b]], zeros16)
        ids = idx_v[pl.ds(blk * _BLK, _LANES)]
        plsc.store_scatter(bufs[b], [lane, ids], ones16)
        pending[b] = ids
        handles[b] = pltpu.async_copy(
            bufs[b],
            out_hbm.at[pl.ds(row_base + blk * _BLK, _BLK), :],
            sems[b],
        )
    for h in handles:
        if h is not None:
            h.wait()


@jax.jit
def _onehot_sc(X_in):
    mesh = plsc.VectorSubcoreMesh(core_axis_name="c", subcore_axis_name="s")
    fn = functools.partial(
        pl.kernel,
        mesh=mesh,
        out_type=jax.ShapeDtypeStruct((_BATCH, _DEPTH), jnp.float32),
        scratch_types=[
            pltpu.VMEM((_ROWS_PER_W,), jnp.int32),
        ]
        + [pltpu.VMEM((_BLK, _DEPTH), jnp.float32) for _ in range(_NBUF)]
        + [pltpu.SemaphoreType.DMA for _ in range(_NBUF)],
        compiler_params=pltpu.CompilerParams(needs_layout_passes=False),
    )(_onehot_body)
    return fn(X_in)


def kernel(X_in, ones):
    del ones
    return _onehot_sc(X_in)


# speedup vs baseline: 4.2726x; 2.2922x over previous
import functools

import jax
import jax.numpy as jnp
from jax import lax
from jax.experimental import pallas as pl
from jax.experimental.pallas import tpu as pltpu
from jax.experimental.pallas import tpu_sc as plsc

_DEPTH = 1000
_BATCH = 16384

_NC = 2
_NS = 16
_NW = _NC * _NS
_LANES = 16

_COLS_PER_W = _BATCH // _NW
_CCHUNKS = _COLS_PER_W // _LANES
_RBLK = 40
_NBLK = _DEPTH // _RBLK
_NBUF = 2


def _onehot_body(idx_hbm, out_hbm, idx_v, b0, b1, s0, s1):
    bufs = [b0, b1]
    sems = [s0, s1]
    wid = lax.axis_index("s") * _NC + lax.axis_index("c")
    col_base = wid * _COLS_PER_W

    zeros16 = jnp.zeros((_LANES,), jnp.float32)
    ones16 = jnp.ones((_LANES,), jnp.float32)
    lane = lax.iota(jnp.int32, _LANES)

    for buf in bufs:
        def _zrow(r, _, buf=buf):
            for u in range(_CCHUNKS):
                buf[r, pl.ds(u * _LANES, _LANES)] = zeros16
            return _

        lax.fori_loop(0, _RBLK, _zrow, None)

    pltpu.sync_copy(idx_hbm.at[pl.ds(col_base, _COLS_PER_W)], idx_v)

    def _scatter_block(buf, r0, val16):
        def _chunk(c, _):
            ids = idx_v[pl.ds(c * _LANES, _LANES)]
            rows = ids - r0
            cols = c * _LANES + lane
            mask = (ids >= r0) & (ids < r0 + _RBLK)
            plsc.store_scatter(buf, [rows, cols], val16, mask=mask)
            return _

        lax.fori_loop(0, _CCHUNKS, _chunk, None)

    handles = [None] * _NBUF
    pending = [None] * _NBUF
    for blk in range(_NBLK):
        b = blk % _NBUF
        r0 = blk * _RBLK
        if handles[b] is not None:
            handles[b].wait()
            _scatter_block(bufs[b], pending[b], zeros16)
        _scatter_block(bufs[b], r0, ones16)
        pending[b] = r0
        handles[b] = pltpu.async_copy(
            bufs[b],
            out_hbm.at[pl.ds(r0, _RBLK), pl.ds(col_base, _COLS_PER_W)],
            sems[b],
        )
    for h in handles:
        if h is not None:
            h.wait()


@jax.jit
def _onehot_sc(X_in):
    mesh = plsc.VectorSubcoreMesh(core_axis_name="c", subcore_axis_name="s")
    fn = functools.partial(
        pl.kernel,
        mesh=mesh,
        out_type=jax.ShapeDtypeStruct((_DEPTH, _BATCH), jnp.float32),
        scratch_types=[
            pltpu.VMEM((_COLS_PER_W,), jnp.int32),
        ]
        + [pltpu.VMEM((_RBLK, _COLS_PER_W), jnp.float32) for _ in range(_NBUF)]
        + [pltpu.SemaphoreType.DMA for _ in range(_NBUF)],
        compiler_params=pltpu.CompilerParams(needs_layout_passes=False),
    )(_onehot_body)
    return _onehot_sc_transpose(fn(X_in))


def _onehot_sc_transpose(x):
    return x.T


def kernel(X_in, ones):
    del ones
    return _onehot_sc(X_in)


# speedup vs baseline: 4.2774x; 1.0011x over previous
import functools

import jax
import jax.numpy as jnp
from jax import lax
from jax.experimental import pallas as pl
from jax.experimental.pallas import tpu as pltpu
from jax.experimental.pallas import tpu_sc as plsc

_DEPTH = 1000
_BATCH = 16384

_NC = 2
_NS = 16
_NW = _NC * _NS
_LANES = 16

_COLS_PER_W = _BATCH // _NW
_CCHUNKS = _COLS_PER_W // _LANES
_RBLK = 40
_NBLK = _DEPTH // _RBLK
_NBUF = 2


def _onehot_body(idx_hbm, out_hbm, idx_v, b0, b1, s0, s1):
    bufs = [b0, b1]
    sems = [s0, s1]
    wid = lax.axis_index("s") * _NC + lax.axis_index("c")
    col_base = wid * _COLS_PER_W

    zeros16 = jnp.zeros((_LANES,), jnp.float32)
    ones16 = jnp.ones((_LANES,), jnp.float32)
    lane = lax.iota(jnp.int32, _LANES)

    for buf in bufs:
        def _zrow(r, _, buf=buf):
            for u in range(_CCHUNKS):
                buf[r, pl.ds(u * _LANES, _LANES)] = zeros16
            return _

        lax.fori_loop(0, _RBLK, _zrow, None)

    pltpu.sync_copy(idx_hbm.at[pl.ds(col_base, _COLS_PER_W)], idx_v)

    def _scatter_block(buf, r0, val16):
        def _chunk(c, _):
            ids = idx_v[pl.ds(c * _LANES, _LANES)]
            rows = ids - r0
            cols = c * _LANES + lane
            mask = (ids >= r0) & (ids < r0 + _RBLK)
            plsc.store_scatter(buf, [rows, cols], val16, mask=mask)
            return _

        lax.fori_loop(0, _CCHUNKS, _chunk, None)

    handles = [None] * _NBUF
    pending = [None] * _NBUF
    for blk in range(_NBLK):
        b = blk % _NBUF
        r0 = blk * _RBLK
        if handles[b] is not None:
            handles[b].wait()
            _scatter_block(bufs[b], pending[b], zeros16)
        _scatter_block(bufs[b], r0, ones16)
        pending[b] = r0
        handles[b] = pltpu.async_copy(
            bufs[b],
            out_hbm.at[pl.ds(r0, _RBLK), pl.ds(col_base, _COLS_PER_W)],
            sems[b],
        )
    for h in handles:
        if h is not None:
            h.wait()


@jax.jit
def _onehot_sc(X_in):
    mesh = plsc.VectorSubcoreMesh(core_axis_name="c", subcore_axis_name="s")
    fn = functools.partial(
        pl.kernel,
        mesh=mesh,
        out_type=jax.ShapeDtypeStruct((_DEPTH, _BATCH), jnp.float32),
        scratch_types=[
            pltpu.VMEM((_COLS_PER_W,), jnp.int32),
        ]
        + [pltpu.VMEM((_RBLK, _COLS_PER_W), jnp.float32) for _ in range(_NBUF)]
        + [pltpu.SemaphoreType.DMA for _ in range(_NBUF)],
        compiler_params=pltpu.CompilerParams(
            needs_layout_passes=False,
            skip_device_barrier=True,
            disable_bounds_checks=True,
            disable_semaphore_checks=True,
        ),
    )(_onehot_body)
    return _onehot_sc_transpose(fn(X_in))


def _onehot_sc_transpose(x):
    return x.T


def kernel(X_in, ones):
    del ones
    return _onehot_sc(X_in)
